# baseline (device time: 234294 ns/iter reference)
import jax
import jax.numpy as jnp
from jax import lax
from jax.experimental import pallas as pl
from jax.experimental.pallas import tpu as pltpu

NY, NZ = 4, 4
NYZ = NY * NZ


def kernel(Q, K, V):
    B, SKV, H, D = K.shape
    scale = D ** -0.5
    assert B == NYZ

    def _tree(x, op):
        while x.shape[0] > H:
            half = x.shape[0] // 2
            x = op(x[:half], x[half:])
        return x

    def body(q_ref, k_hbm, v_hbm, o_ref,
             kbuf, vbuf, obuf, accb, mb, lb, racc, rm, rl,
             ksem, vsem, xsend, xrecv, bss, brs):
        my_x = lax.axis_index("x")
        my_y = lax.axis_index("y")
        my_z = lax.axis_index("z")
        my_yz = my_y * NZ + my_z
        peer_x = (1 - my_x, my_y, my_z)

        kcp = pltpu.make_async_copy(k_hbm.at[my_yz], kbuf, ksem)
        vcp = pltpu.make_async_copy(v_hbm.at[my_yz], vbuf, vsem)
        kcp.start()
        vcp.start()

        bsem = pltpu.get_barrier_semaphore()
        pl.semaphore_signal(
            bsem, inc=1, device_id=peer_x,
            device_id_type=pl.DeviceIdType.MESH,
        )
        for dy in range(NY):
            for dz in range(NZ):
                dyz = dy * NZ + dz

                @pl.when(dyz != my_yz)
                def _():
                    pl.semaphore_signal(
                        bsem, inc=1, device_id=(my_x, dy, dz),
                        device_id_type=pl.DeviceIdType.MESH,
                    )
        pl.semaphore_wait(bsem, NYZ)

        q2 = q_ref[my_yz, 0]

        kcp.wait()
        kf = kbuf[...].reshape(SKV * H, D)
        c = lax.dot_general(
            kf, q2, (((1,), (1,)), ((), ())),
            preferred_element_type=jnp.float32,
        )
        rowmod = lax.broadcasted_iota(jnp.int32, (SKV * H, H), 0) % H
        col = lax.broadcasted_iota(jnp.int32, (SKV * H, H), 1)
        cm = jnp.where(rowmod == col, c * scale, -jnp.inf)
        r16m = _tree(cm, jnp.maximum)
        mrow = jnp.max(r16m, axis=0, keepdims=True)
        m_col = jnp.max(r16m, axis=1, keepdims=True)
        p2 = jnp.exp(cm - mrow)
        l_col = jnp.sum(_tree(p2, jnp.add), axis=1, keepdims=True)
        p_flat = jnp.sum(p2, axis=1, keepdims=True)

        vcp.wait()
        vf = vbuf[...].reshape(SKV * H, D)
        acc = _tree(vf * p_flat, jnp.add)

        accb[...] = acc
        mb[...] = m_col
        lb[...] = l_col

        rdmas = []
        for i, (src, dst) in enumerate([(accb, racc), (mb, rm), (lb, rl)]):
            rdma = pltpu.make_async_remote_copy(
                src_ref=src,
                dst_ref=dst,
                send_sem=xsend.at[i],
                recv_sem=xrecv.at[i],
                device_id=peer_x,
                device_id_type=pl.DeviceIdType.MESH,
            )
            rdma.start()
            rdmas.append(rdma)
        for rdma in rdmas:
            rdma.wait()

        m_r = rm[...]
        l_r = rl[...]
        mn = jnp.maximum(m_col, m_r)
        ea = jnp.exp(m_col - mn)
        eb = jnp.exp(m_r - mn)
        ln = l_col * ea + l_r * eb
        obuf[my_yz] = (acc * ea + racc[...] * eb) / ln

        for dy in range(NY):
            for dz in range(NZ):
                dyz = dy * NZ + dz

                @pl.when(dyz != my_yz)
                def _():
                    rdma = pltpu.make_async_remote_copy(
                        src_ref=obuf.at[my_yz],
                        dst_ref=obuf.at[my_yz],
                        send_sem=bss.at[dyz],
                        recv_sem=brs.at[my_yz],
                        device_id=(my_x, dy, dz),
                        device_id_type=pl.DeviceIdType.MESH,
                    )
                    rdma.start()

        for j in range(NYZ):

            @pl.when(j != my_yz)
            def _():
                rcv = pltpu.make_async_remote_copy(
                    src_ref=obuf.at[j],
                    dst_ref=obuf.at[j],
                    send_sem=bss.at[j],
                    recv_sem=brs.at[j],
                    device_id=peer_x,
                    device_id_type=pl.DeviceIdType.MESH,
                )
                rcv.wait_recv()
                snd = pltpu.make_async_remote_copy(
                    src_ref=obuf.at[my_yz],
                    dst_ref=obuf.at[j],
                    send_sem=bss.at[j],
                    recv_sem=brs.at[j],
                    device_id=peer_x,
                    device_id_type=pl.DeviceIdType.MESH,
                )
                snd.wait_send()

        o_ref[:, 0, :, :] = obuf[...]

    return pl.pallas_call(
        body,
        in_specs=[
            pl.BlockSpec(memory_space=pltpu.VMEM),
            pl.BlockSpec(memory_space=pltpu.MemorySpace.HBM),
            pl.BlockSpec(memory_space=pltpu.MemorySpace.HBM),
        ],
        out_specs=pl.BlockSpec(memory_space=pltpu.VMEM),
        out_shape=jax.ShapeDtypeStruct((B, 1, H, D), jnp.float32),
        scratch_shapes=[
            pltpu.VMEM((SKV, H, D), jnp.float32),
            pltpu.VMEM((SKV, H, D), jnp.float32),
            pltpu.VMEM((B, H, D), jnp.float32),
            pltpu.VMEM((H, D), jnp.float32),
            pltpu.VMEM((H, 1), jnp.float32),
            pltpu.VMEM((H, 1), jnp.float32),
            pltpu.VMEM((H, D), jnp.float32),
            pltpu.VMEM((H, 1), jnp.float32),
            pltpu.VMEM((H, 1), jnp.float32),
            pltpu.SemaphoreType.DMA,
            pltpu.SemaphoreType.DMA,
            pltpu.SemaphoreType.DMA((3,)),
            pltpu.SemaphoreType.DMA((3,)),
            pltpu.SemaphoreType.DMA((NYZ,)),
            pltpu.SemaphoreType.DMA((NYZ,)),
        ],
        compiler_params=pltpu.CompilerParams(collective_id=0),
    )(Q, K, V)


# device time: 24302 ns/iter; 9.6409x vs baseline; 9.6409x over previous
import jax
import jax.numpy as jnp
from jax import lax
from jax.experimental import pallas as pl
from jax.experimental.pallas import tpu as pltpu

NY, NZ = 4, 4
NYZ = NY * NZ


def kernel(Q, K, V):
    B, SKV, H, D = K.shape
    HD = H * D
    scale = D ** -0.5
    assert B == NYZ

    my_b = lax.axis_index("y") * NZ + lax.axis_index("z")
    qb = lax.dynamic_slice_in_dim(Q, my_b, 1, 0).reshape(HD, 1)
    k2 = lax.dynamic_slice_in_dim(K, my_b, 1, 0).reshape(SKV, HD)
    v2 = lax.dynamic_slice_in_dim(V, my_b, 1, 0).reshape(SKV, HD)

    def body(q_ref, k_ref, v_ref, o_ref,
             obuf, accb, mb, lb, racc, rm, rl,
             xsend, xrecv, bss, brs):
        my_x = lax.axis_index("x")
        my_y = lax.axis_index("y")
        my_z = lax.axis_index("z")
        my_yz = my_y * NZ + my_z
        peer_x = (1 - my_x, my_y, my_z)

        bsem = pltpu.get_barrier_semaphore()
        pl.semaphore_signal(
            bsem, inc=1, device_id=peer_x,
            device_id_type=pl.DeviceIdType.MESH,
        )
        for dy in range(NY):
            for dz in range(NZ):
                dyz = dy * NZ + dz

                @pl.when(dyz != my_yz)
                def _():
                    pl.semaphore_signal(
                        bsem, inc=1, device_id=(my_x, dy, dz),
                        device_id_type=pl.DeviceIdType.MESH,
                    )
        pl.semaphore_wait(bsem, NYZ)

        row_h = lax.broadcasted_iota(jnp.int32, (HD, H), 0) // D
        col_h = lax.broadcasted_iota(jnp.int32, (HD, H), 1)
        qmask = row_h == col_h
        prow = lax.broadcasted_iota(jnp.int32, (H, HD), 0)
        pcol = lax.broadcasted_iota(jnp.int32, (H, HD), 1) // D
        pmask = prow == pcol

        qf = q_ref[...]
        qd = jnp.where(qmask, jnp.broadcast_to(qf, (HD, H)), 0.0)

        sm = lax.dot_general(
            k_ref[...], qd, (((1,), (0,)), ((), ())),
            preferred_element_type=jnp.float32,
        ) * scale
        m = jnp.max(sm, axis=0, keepdims=True)
        p = jnp.exp(sm - m)
        l = jnp.sum(p, axis=0, keepdims=True)

        ptv = lax.dot_general(
            p, v_ref[...], (((0,), (0,)), ((), ())),
            preferred_element_type=jnp.float32,
        )
        acc = jnp.sum(
            jnp.where(pmask, ptv, 0.0), axis=0, keepdims=True
        )

        accb[...] = acc
        mb[...] = m
        lb[...] = l

        rdmas = []
        for i, (src, dst) in enumerate([(accb, racc), (mb, rm), (lb, rl)]):
            rdma = pltpu.make_async_remote_copy(
                src_ref=src,
                dst_ref=dst,
                send_sem=xsend.at[i],
                recv_sem=xrecv.at[i],
                device_id=peer_x,
                device_id_type=pl.DeviceIdType.MESH,
            )
            rdma.start()
            rdmas.append(rdma)
        for rdma in rdmas:
            rdma.wait()

        m_r = rm[...]
        l_r = rl[...]
        mn = jnp.maximum(m, m_r)
        ea = jnp.exp(m - mn)
        eb = jnp.exp(m_r - mn)
        ln = l * ea + l_r * eb
        emat = jnp.where(pmask, 1.0, 0.0)
        dn = (((1,), (0,)), ((), ()))
        eae = lax.dot_general(ea, emat, dn,
                              preferred_element_type=jnp.float32)
        ebe = lax.dot_general(eb, emat, dn,
                              preferred_element_type=jnp.float32)
        lne = lax.dot_general(ln, emat, dn,
                              preferred_element_type=jnp.float32)
        obuf[my_yz] = (acc * eae + racc[...] * ebe) / lne

        for dy in range(NY):
            for dz in range(NZ):
                dyz = dy * NZ + dz

                @pl.when(dyz != my_yz)
                def _():
                    rdma = pltpu.make_async_remote_copy(
                        src_ref=obuf.at[my_yz],
                        dst_ref=obuf.at[my_yz],
                        send_sem=bss.at[dyz],
                        recv_sem=brs.at[my_yz],
                        device_id=(my_x, dy, dz),
                        device_id_type=pl.DeviceIdType.MESH,
                    )
                    rdma.start()

        for j in range(NYZ):

            @pl.when(j != my_yz)
            def _():
                rcv = pltpu.make_async_remote_copy(
                    src_ref=obuf.at[j],
                    dst_ref=obuf.at[j],
                    send_sem=bss.at[j],
                    recv_sem=brs.at[j],
                    device_id=peer_x,
                    device_id_type=pl.DeviceIdType.MESH,
                )
                rcv.wait_recv()
                snd = pltpu.make_async_remote_copy(
                    src_ref=obuf.at[my_yz],
                    dst_ref=obuf.at[j],
                    send_sem=bss.at[j],
                    recv_sem=brs.at[j],
                    device_id=peer_x,
                    device_id_type=pl.DeviceIdType.MESH,
                )
                snd.wait_send()

        o_ref[...] = obuf[...]

    out = pl.pallas_call(
        body,
        in_specs=[
            pl.BlockSpec(memory_space=pltpu.VMEM),
            pl.BlockSpec(memory_space=pltpu.VMEM),
            pl.BlockSpec(memory_space=pltpu.VMEM),
        ],
        out_specs=pl.BlockSpec(memory_space=pltpu.VMEM),
        out_shape=jax.ShapeDtypeStruct((B, 1, HD), jnp.float32),
        scratch_shapes=[
            pltpu.VMEM((B, 1, HD), jnp.float32),
            pltpu.VMEM((1, HD), jnp.float32),
            pltpu.VMEM((1, H), jnp.float32),
            pltpu.VMEM((1, H), jnp.float32),
            pltpu.VMEM((1, HD), jnp.float32),
            pltpu.VMEM((1, H), jnp.float32),
            pltpu.VMEM((1, H), jnp.float32),
            pltpu.SemaphoreType.DMA((3,)),
            pltpu.SemaphoreType.DMA((3,)),
            pltpu.SemaphoreType.DMA((NYZ,)),
            pltpu.SemaphoreType.DMA((NYZ,)),
        ],
        compiler_params=pltpu.CompilerParams(collective_id=0),
    )(qb, k2, v2)
    return out.reshape(B, 1, H, D)


# device time: 22375 ns/iter; 10.4712x vs baseline; 1.0861x over previous
import jax
import jax.numpy as jnp
from jax import lax
from jax.experimental import pallas as pl
from jax.experimental.pallas import tpu as pltpu

NY, NZ = 4, 4
NYZ = NY * NZ


def kernel(Q, K, V):
    B, SKV, H, D = K.shape
    HD = H * D
    scale = D ** -0.5
    assert B == NYZ

    my_b = lax.axis_index("y") * NZ + lax.axis_index("z")
    bf16 = jnp.bfloat16
    qb = lax.dynamic_slice_in_dim(Q, my_b, 1, 0).reshape(HD, 1)
    k2 = lax.dynamic_slice_in_dim(K, my_b, 1, 0).reshape(SKV, HD).astype(bf16)
    v2 = lax.dynamic_slice_in_dim(V, my_b, 1, 0).reshape(SKV, HD).astype(bf16)

    def body(q_ref, k_ref, v_ref, o_ref,
             obuf, accb, mb, lb, racc, rm, rl,
             xsend, xrecv, bss, brs):
        my_x = lax.axis_index("x")
        my_y = lax.axis_index("y")
        my_z = lax.axis_index("z")
        my_yz = my_y * NZ + my_z
        peer_x = (1 - my_x, my_y, my_z)

        bsem = pltpu.get_barrier_semaphore()
        pl.semaphore_signal(
            bsem, inc=1, device_id=peer_x,
            device_id_type=pl.DeviceIdType.MESH,
        )
        for dy in range(NY):
            for dz in range(NZ):
                dyz = dy * NZ + dz

                @pl.when(dyz != my_yz)
                def _():
                    pl.semaphore_signal(
                        bsem, inc=1, device_id=(my_x, dy, dz),
                        device_id_type=pl.DeviceIdType.MESH,
                    )
        pl.semaphore_wait(bsem, NYZ)

        row_h = lax.broadcasted_iota(jnp.int32, (HD, H), 0) // D
        col_h = lax.broadcasted_iota(jnp.int32, (HD, H), 1)
        qmask = row_h == col_h
        prow = lax.broadcasted_iota(jnp.int32, (H, HD), 0)
        pcol = lax.broadcasted_iota(jnp.int32, (H, HD), 1) // D
        pmask = prow == pcol

        qf = q_ref[...]
        qd = jnp.where(
            qmask, jnp.broadcast_to(qf, (HD, H)), 0.0
        ).astype(jnp.bfloat16)

        sm = lax.dot_general(
            k_ref[...], qd, (((1,), (0,)), ((), ())),
            preferred_element_type=jnp.float32,
        ) * scale
        m = jnp.max(sm, axis=0, keepdims=True)
        p = jnp.exp(sm - m)
        l = jnp.sum(p, axis=0, keepdims=True)

        ptv = lax.dot_general(
            p.astype(jnp.bfloat16), v_ref[...], (((0,), (0,)), ((), ())),
            preferred_element_type=jnp.float32,
        )
        acc = jnp.sum(
            jnp.where(pmask, ptv, 0.0), axis=0, keepdims=True
        )

        accb[...] = acc
        mb[...] = m
        lb[...] = l

        rdmas = []
        for i, (src, dst) in enumerate([(accb, racc), (mb, rm), (lb, rl)]):
            rdma = pltpu.make_async_remote_copy(
                src_ref=src,
                dst_ref=dst,
                send_sem=xsend.at[i],
                recv_sem=xrecv.at[i],
                device_id=peer_x,
                device_id_type=pl.DeviceIdType.MESH,
            )
            rdma.start()
            rdmas.append(rdma)
        for rdma in rdmas:
            rdma.wait()

        m_r = rm[...]
        l_r = rl[...]
        mn = jnp.maximum(m, m_r)
        ea = jnp.exp(m - mn)
        eb = jnp.exp(m_r - mn)
        ln = l * ea + l_r * eb
        emat = jnp.where(pmask, 1.0, 0.0)
        dn = (((1,), (0,)), ((), ()))
        eae = lax.dot_general(ea, emat, dn,
                              preferred_element_type=jnp.float32)
        ebe = lax.dot_general(eb, emat, dn,
                              preferred_element_type=jnp.float32)
        lne = lax.dot_general(ln, emat, dn,
                              preferred_element_type=jnp.float32)
        obuf[my_yz] = (acc * eae + racc[...] * ebe) / lne

        for dy in range(NY):
            for dz in range(NZ):
                dyz = dy * NZ + dz

                @pl.when(dyz != my_yz)
                def _():
                    rdma = pltpu.make_async_remote_copy(
                        src_ref=obuf.at[my_yz],
                        dst_ref=obuf.at[my_yz],
                        send_sem=bss.at[dyz],
                        recv_sem=brs.at[my_yz],
                        device_id=(my_x, dy, dz),
                        device_id_type=pl.DeviceIdType.MESH,
                    )
                    rdma.start()

        for j in range(NYZ):

            @pl.when(j != my_yz)
            def _():
                rcv = pltpu.make_async_remote_copy(
                    src_ref=obuf.at[j],
                    dst_ref=obuf.at[j],
                    send_sem=bss.at[j],
                    recv_sem=brs.at[j],
                    device_id=peer_x,
                    device_id_type=pl.DeviceIdType.MESH,
                )
                rcv.wait_recv()
                snd = pltpu.make_async_remote_copy(
                    src_ref=obuf.at[my_yz],
                    dst_ref=obuf.at[j],
                    send_sem=bss.at[j],
                    recv_sem=brs.at[j],
                    device_id=peer_x,
                    device_id_type=pl.DeviceIdType.MESH,
                )
                snd.wait_send()

        o_ref[...] = obuf[...]

    out = pl.pallas_call(
        body,
        in_specs=[
            pl.BlockSpec(memory_space=pltpu.VMEM),
            pl.BlockSpec(memory_space=pltpu.VMEM),
            pl.BlockSpec(memory_space=pltpu.VMEM),
        ],
        out_specs=pl.BlockSpec(memory_space=pltpu.VMEM),
        out_shape=jax.ShapeDtypeStruct((B, 1, HD), jnp.float32),
        scratch_shapes=[
            pltpu.VMEM((B, 1, HD), jnp.float32),
            pltpu.VMEM((1, HD), jnp.float32),
            pltpu.VMEM((1, H), jnp.float32),
            pltpu.VMEM((1, H), jnp.float32),
            pltpu.VMEM((1, HD), jnp.float32),
            pltpu.VMEM((1, H), jnp.float32),
            pltpu.VMEM((1, H), jnp.float32),
            pltpu.SemaphoreType.DMA((3,)),
            pltpu.SemaphoreType.DMA((3,)),
            pltpu.SemaphoreType.DMA((NYZ,)),
            pltpu.SemaphoreType.DMA((NYZ,)),
        ],
        compiler_params=pltpu.CompilerParams(collective_id=0),
    )(qb, k2, v2)
    return out.reshape(B, 1, H, D)
